# baseline (speedup 1.0000x reference)
import functools

import jax
import jax.numpy as jnp
from jax import lax
from jax.experimental import pallas as pl
from jax.experimental.pallas import tpu as pltpu
from jax.experimental.pallas import tpu_sc as plsc

N_TOK = 8192
C_IN = 192
D = 64
K = 8192
TMT = 1024
KB = 1024


def _tc_body(z_ref, w_ref, b_ref, e_ref, idx_ref, loss_ref, en_ref):
    i = pl.program_id(0)

    @pl.when(i == 0)
    def _():
        e = e_ref[...]
        en_ref[...] = jnp.sum(e * e, axis=1, keepdims=True)

    zc = z_ref[0]
    ze = jnp.dot(w_ref[...], zc,
                 preferred_element_type=jnp.float32) + b_ref[...]
    fnorm = jnp.sum(ze * ze, axis=0, keepdims=True)
    zem2 = ze * (-2.0)

    best = jnp.full((1, TMT), jnp.inf, jnp.float32)
    bidxf = jnp.zeros((1, TMT), jnp.float32)
    iot = lax.broadcasted_iota(jnp.int32, (KB, TMT), 0).astype(jnp.float32)
    for j in range(K // KB):
        ec = e_ref[pl.ds(j * KB, KB), :]
        s2 = lax.dot_general(ec, zem2, (((1,), (0,)), ((), ())),
                             preferred_element_type=jnp.float32)
        en = en_ref[pl.ds(j * KB, KB), :]
        dch = (fnorm + s2) + en
        lmin = jnp.min(dch, axis=0, keepdims=True)
        lidx = jnp.min(jnp.where(dch == lmin, iot, float(K)),
                       axis=0, keepdims=True) + float(j * KB)
        take = lmin < best
        best = jnp.where(take, lmin, best)
        bidxf = jnp.where(take, lidx, bidxf)

    idx_ref[...] = bidxf.astype(jnp.int32).reshape(TMT)
    prev = jnp.where(i == 0, 0.0, loss_ref[...])
    loss_ref[...] = prev + jnp.sum(best).reshape(1, 1)


def _distance_argmin(z3, w, b2, embed):
    n = z3.shape[0] * TMT
    return pl.pallas_call(
        _tc_body,
        grid=(n // TMT,),
        in_specs=[
            pl.BlockSpec((1, C_IN, TMT), lambda i: (i, 0, 0)),
            pl.BlockSpec((D, C_IN), lambda i: (0, 0)),
            pl.BlockSpec((D, 1), lambda i: (0, 0)),
            pl.BlockSpec((K, D), lambda i: (0, 0)),
        ],
        out_specs=[
            pl.BlockSpec((TMT,), lambda i: (i,)),
            pl.BlockSpec((1, 1), lambda i: (0, 0)),
        ],
        out_shape=[
            jax.ShapeDtypeStruct((n,), jnp.int32),
            jax.ShapeDtypeStruct((1, 1), jnp.float32),
        ],
        scratch_shapes=[pltpu.VMEM((K, 1), jnp.float32)],
    )(z3, w, b2, embed)


@functools.cache
def _make_gather(n_tok):
    info = plsc.get_sparse_core_info()
    nw = info.num_cores * info.num_subcores
    ch = 128
    rounds = n_tok // (nw * ch)
    mesh = plsc.VectorSubcoreMesh(core_axis_name="c", subcore_axis_name="s")

    @functools.partial(
        pl.kernel, mesh=mesh,
        compiler_params=pltpu.CompilerParams(use_tc_tiling_on_sc=False),
        out_type=jax.ShapeDtypeStruct((n_tok, D), jnp.float32),
        scratch_types=[
            pltpu.VMEM((ch,), jnp.int32),
            pltpu.VMEM((ch, D), jnp.float32),
            pltpu.SemaphoreType.DMA,
        ],
    )
    def gather(table_hbm, idx_hbm, out_hbm, idx_v, rows_v, sem):
        wid = lax.axis_index("s") * info.num_cores + lax.axis_index("c")
        for g in range(rounds):
            base = (g * nw + wid) * ch
            pltpu.sync_copy(idx_hbm.at[pl.ds(base, ch)], idx_v)
            pltpu.async_copy(table_hbm.at[idx_v], rows_v, sem).wait()
            pltpu.sync_copy(rows_v, out_hbm.at[pl.ds(base, ch)])

    return gather


def kernel(z, proj_w, proj_b, embed):
    B, C, H, W = z.shape
    z3 = z.reshape(B, C_IN, H * W)
    b2 = proj_b.reshape(D, 1)
    hb = B // 2
    gather = _make_gather(hb * TMT)
    idx0, loss0 = _distance_argmin(z3[:hb], proj_w, b2, embed)
    z_q0 = gather(embed, idx0)
    idx1, loss1 = _distance_argmin(z3[hb:], proj_w, b2, embed)
    z_q1 = gather(embed, idx1)
    z_q_flat = jnp.concatenate([z_q0, z_q1], axis=0).reshape(B, H, W, D)
    z_q_st = z_q_flat.transpose(0, 3, 1, 2)
    latent_loss = ((loss0 + loss1) * (12.5 / (N_TOK * D))).reshape(())
    z_q_ind = jnp.concatenate([idx0, idx1]).reshape(B, H, W)
    return (z_q_st, z_q_flat, latent_loss, z_q_ind)


# speedup vs baseline: 1.0607x; 1.0607x over previous
import functools

import jax
import jax.numpy as jnp
from jax import lax
from jax.experimental import pallas as pl
from jax.experimental.pallas import tpu as pltpu
from jax.experimental.pallas import tpu_sc as plsc

N_TOK = 8192
C_IN = 192
D = 64
K = 8192
TMT = 1024
KB = 1024


def _tc_body(z_ref, w_ref, b_ref, e_ref, idx_ref, loss_ref, en_ref):
    i = pl.program_id(0)

    @pl.when(i == 0)
    def _():
        e = e_ref[...]
        en_ref[...] = jnp.sum(e * e, axis=1, keepdims=True)

    zc = z_ref[0]
    ze = jnp.dot(w_ref[...], zc,
                 preferred_element_type=jnp.float32) + b_ref[...]
    fnorm = jnp.sum(ze * ze, axis=0, keepdims=True)
    zem2 = ze * (-2.0)

    best = jnp.full((1, TMT), jnp.inf, jnp.float32)
    bidxf = jnp.zeros((1, TMT), jnp.float32)
    iot = lax.broadcasted_iota(jnp.int32, (KB, TMT), 0).astype(jnp.float32)
    for j in range(K // KB):
        ec = e_ref[pl.ds(j * KB, KB), :]
        s2 = lax.dot_general(ec, zem2, (((1,), (0,)), ((), ())),
                             preferred_element_type=jnp.float32)
        en = en_ref[pl.ds(j * KB, KB), :]
        dch = (fnorm + s2) + en
        lmin = jnp.min(dch, axis=0, keepdims=True)
        lidx = jnp.min(jnp.where(dch == lmin, iot, float(K)),
                       axis=0, keepdims=True) + float(j * KB)
        take = lmin < best
        best = jnp.where(take, lmin, best)
        bidxf = jnp.where(take, lidx, bidxf)

    idx_ref[...] = bidxf.astype(jnp.int32).reshape(TMT)
    prev = jnp.where(i == 0, 0.0, loss_ref[...])
    loss_ref[...] = prev + jnp.sum(best).reshape(1, 1)


def _distance_argmin(z3, w, b2, embed):
    n = z3.shape[0] * TMT
    return pl.pallas_call(
        _tc_body,
        grid=(n // TMT,),
        in_specs=[
            pl.BlockSpec((1, C_IN, TMT), lambda i: (i, 0, 0)),
            pl.BlockSpec((D, C_IN), lambda i: (0, 0)),
            pl.BlockSpec((D, 1), lambda i: (0, 0)),
            pl.BlockSpec((K, D), lambda i: (0, 0)),
        ],
        out_specs=[
            pl.BlockSpec((TMT,), lambda i: (i,)),
            pl.BlockSpec((1, 1), lambda i: (0, 0)),
        ],
        out_shape=[
            jax.ShapeDtypeStruct((n,), jnp.int32),
            jax.ShapeDtypeStruct((1, 1), jnp.float32),
        ],
        scratch_shapes=[pltpu.VMEM((K, 1), jnp.float32)],
    )(z3, w, b2, embed)


@functools.cache
def _make_gather(n_tok):
    info = plsc.get_sparse_core_info()
    nw = info.num_cores * info.num_subcores
    ch = 128
    rounds = n_tok // (nw * ch)
    mesh = plsc.VectorSubcoreMesh(core_axis_name="c", subcore_axis_name="s")

    @functools.partial(
        pl.kernel, mesh=mesh,
        compiler_params=pltpu.CompilerParams(use_tc_tiling_on_sc=False),
        out_type=jax.ShapeDtypeStruct((n_tok, D), jnp.float32),
        scratch_types=[
            pltpu.VMEM((rounds, ch), jnp.int32),
            pltpu.VMEM((rounds, ch, D), jnp.float32),
            pltpu.SemaphoreType.DMA,
        ],
    )
    def gather(table_hbm, idx_hbm, out_hbm, idx_v, rows_v, sem):
        wid = lax.axis_index("s") * info.num_cores + lax.axis_index("c")
        for g in range(rounds):
            base = (g * nw + wid) * ch
            pltpu.sync_copy(idx_hbm.at[pl.ds(base, ch)], idx_v.at[g])
        copies = []
        for g in range(rounds):
            copies.append(pltpu.async_copy(
                table_hbm.at[idx_v.at[g]], rows_v.at[g], sem))
        for g in range(rounds):
            base = (g * nw + wid) * ch
            copies[g].wait()
            pltpu.sync_copy(rows_v.at[g], out_hbm.at[pl.ds(base, ch)])

    return gather


def kernel(z, proj_w, proj_b, embed):
    B, C, H, W = z.shape
    z3 = z.reshape(B, C_IN, H * W)
    idx, loss_acc = _distance_argmin(
        z3, proj_w, proj_b.reshape(D, 1), embed)
    z_q = _make_gather(N_TOK)(embed, idx)
    z_q_flat = z_q.reshape(B, H, W, D)
    z_q_st = z_q_flat.transpose(0, 3, 1, 2)
    latent_loss = (loss_acc * (12.5 / (N_TOK * D))).reshape(())
    z_q_ind = idx.reshape(B, H, W)
    return (z_q_st, z_q_flat, latent_loss, z_q_ind)
